# Initial kernel scaffold; baseline (speedup 1.0000x reference)
import jax
import jax.numpy as jnp
from jax.experimental import pallas as pl


def kernel(x, W1, b1, W2, b2, Wg, att_src, att_dst, bg, Wfc, bfc, edge_index):
    raise NotImplementedError("write your pallas kernel here")


# speedup vs baseline: 1.0149x; 1.0149x over previous
import jax
import jax.numpy as jnp
from jax.experimental import pallas as pl

_N = 10000
_HEADS = 8
_GOUT = 128


def _mm_bias_kernel(x_ref, w_ref, b_ref, o_ref, *, relu):
    acc = jnp.dot(x_ref[...], w_ref[...], preferred_element_type=jnp.float32)
    acc = acc + b_ref[...]
    if relu:
        acc = jnp.maximum(acc, 0.0)
    o_ref[...] = acc


def _matmul_bias(x, W, b, relu=False, block=400):
    n, k = x.shape
    m = W.shape[1]
    pad = (-n) % block
    if pad:
        x = jnp.concatenate([x, jnp.zeros((pad, k), x.dtype)], axis=0)
    npad = x.shape[0]
    b2 = b.reshape(1, m)
    import functools
    out = pl.pallas_call(
        functools.partial(_mm_bias_kernel, relu=relu),
        grid=(npad // block,),
        in_specs=[
            pl.BlockSpec((block, k), lambda i: (i, 0)),
            pl.BlockSpec((k, m), lambda i: (0, 0)),
            pl.BlockSpec((1, m), lambda i: (0, 0)),
        ],
        out_specs=pl.BlockSpec((block, m), lambda i: (i, 0)),
        out_shape=jax.ShapeDtypeStruct((npad, m), jnp.float32),
    )(x, W, b2)
    return out[:n] if pad else out


def _fc_softmax_kernel(p_ref, w_ref, b_ref, o_ref):
    logits = jnp.dot(p_ref[...], w_ref[...], preferred_element_type=jnp.float32)
    logits = logits + b_ref[...]
    mx = jnp.max(logits, axis=-1, keepdims=True)
    ex = jnp.exp(logits - mx)
    o_ref[...] = ex / jnp.sum(ex, axis=-1, keepdims=True)


def _fc_softmax(pooled, Wfc, bfc):
    k = pooled.shape[0]
    m = Wfc.shape[1]
    out = pl.pallas_call(
        _fc_softmax_kernel,
        in_specs=[
            pl.BlockSpec((1, k), lambda: (0, 0)),
            pl.BlockSpec((k, m), lambda: (0, 0)),
            pl.BlockSpec((1, m), lambda: (0, 0)),
        ],
        out_specs=pl.BlockSpec((1, m), lambda: (0, 0)),
        out_shape=jax.ShapeDtypeStruct((1, m), jnp.float32),
    )(pooled.reshape(1, k), Wfc, bfc.reshape(1, m))
    return out[0]


def _gcn(x, src, dst, W, b, dinv_norm, relu_out):
    h = _matmul_bias(x, W, jnp.zeros((W.shape[1],), jnp.float32))
    msg = h[src] * dinv_norm[:, None]
    out = jax.ops.segment_sum(msg, dst, num_segments=_N)
    out = out + b
    if relu_out:
        out = jnp.maximum(out, 0.0)
    return out


def kernel(x, W1, b1, W2, b2, Wg, att_src, att_dst, bg, Wfc, bfc, edge_index):
    loop = jnp.arange(_N, dtype=edge_index.dtype)
    src = jnp.concatenate([edge_index[0], loop])
    dst = jnp.concatenate([edge_index[1], loop])

    ones = jnp.ones(src.shape[0], dtype=jnp.float32)
    deg = jax.ops.segment_sum(ones, dst, num_segments=_N)
    dinv = jnp.where(deg > 0, jax.lax.rsqrt(jnp.maximum(deg, 1e-12)), 0.0)
    norm = dinv[src] * dinv[dst]

    h = _gcn(x, src, dst, W1, b1, norm, relu_out=True)
    h = _gcn(h, src, dst, W2, b2, norm, relu_out=True)

    hg = _matmul_bias(h, Wg, jnp.zeros((_HEADS * _GOUT,), jnp.float32))
    hg = hg.reshape(_N, _HEADS, _GOUT)
    a_src = jnp.sum(hg * att_src[None, :, :], axis=-1)
    a_dst = jnp.sum(hg * att_dst[None, :, :], axis=-1)
    e = jax.nn.leaky_relu(a_src[src] + a_dst[dst], negative_slope=0.2)
    e_max = jax.ops.segment_max(e, dst, num_segments=_N)
    e_max = jnp.where(jnp.isfinite(e_max), e_max, 0.0)
    p = jnp.exp(e - e_max[dst])
    denom = jax.ops.segment_sum(p, dst, num_segments=_N)
    alpha = p / (denom[dst] + 1e-16)
    msg = hg[src] * alpha[:, :, None]
    out = jax.ops.segment_sum(msg, dst, num_segments=_N)
    hga = jnp.maximum(out.reshape(_N, _HEADS * _GOUT) + bg, 0.0)

    pooled = jnp.mean(hga, axis=0)
    return _fc_softmax(pooled, Wfc, bfc)


# speedup vs baseline: 1.0190x; 1.0041x over previous
import jax
import jax.numpy as jnp
from jax.experimental import pallas as pl

_N = 10000
_HEADS = 8
_GOUT = 128


def _mm_bias_kernel(x_ref, w_ref, b_ref, o_ref, *, relu):
    acc = jnp.dot(x_ref[...], w_ref[...], preferred_element_type=jnp.float32)
    acc = acc + b_ref[...]
    if relu:
        acc = jnp.maximum(acc, 0.0)
    o_ref[...] = acc


def _matmul_bias(x, W, b, relu=False, block=400):
    n, k = x.shape
    m = W.shape[1]
    pad = (-n) % block
    if pad:
        x = jnp.concatenate([x, jnp.zeros((pad, k), x.dtype)], axis=0)
    npad = x.shape[0]
    b2 = b.reshape(1, m)
    import functools
    out = pl.pallas_call(
        functools.partial(_mm_bias_kernel, relu=relu),
        grid=(npad // block,),
        in_specs=[
            pl.BlockSpec((block, k), lambda i: (i, 0)),
            pl.BlockSpec((k, m), lambda i: (0, 0)),
            pl.BlockSpec((1, m), lambda i: (0, 0)),
        ],
        out_specs=pl.BlockSpec((block, m), lambda i: (i, 0)),
        out_shape=jax.ShapeDtypeStruct((npad, m), jnp.float32),
    )(x, W, b2)
    return out[:n] if pad else out


def _gat_proj_kernel(x_ref, w_ref, as_ref, ad_ref, hg_ref, asrc_ref, adst_ref):
    hg = jnp.dot(x_ref[...], w_ref[...], preferred_element_type=jnp.float32)
    hg_ref[...] = hg
    asrc_ref[...] = jnp.dot(hg, as_ref[...], preferred_element_type=jnp.float32)
    adst_ref[...] = jnp.dot(hg, ad_ref[...], preferred_element_type=jnp.float32)


def _gat_proj(h, Wg, att_src, att_dst, block=400):
    n, k = h.shape
    m = Wg.shape[1]
    eye = jnp.eye(_HEADS, dtype=jnp.float32)
    As = (att_src[:, :, None] * eye[:, None, :]).reshape(m, _HEADS)
    Ad = (att_dst[:, :, None] * eye[:, None, :]).reshape(m, _HEADS)
    hg, a_src, a_dst = pl.pallas_call(
        _gat_proj_kernel,
        grid=(n // block,),
        in_specs=[
            pl.BlockSpec((block, k), lambda i: (i, 0)),
            pl.BlockSpec((k, m), lambda i: (0, 0)),
            pl.BlockSpec((m, _HEADS), lambda i: (0, 0)),
            pl.BlockSpec((m, _HEADS), lambda i: (0, 0)),
        ],
        out_specs=[
            pl.BlockSpec((block, m), lambda i: (i, 0)),
            pl.BlockSpec((block, _HEADS), lambda i: (i, 0)),
            pl.BlockSpec((block, _HEADS), lambda i: (i, 0)),
        ],
        out_shape=[
            jax.ShapeDtypeStruct((n, m), jnp.float32),
            jax.ShapeDtypeStruct((n, _HEADS), jnp.float32),
            jax.ShapeDtypeStruct((n, _HEADS), jnp.float32),
        ],
    )(h, Wg, As, Ad)
    return hg, a_src, a_dst


def _fc_softmax_kernel(p_ref, w_ref, b_ref, o_ref):
    logits = jnp.dot(p_ref[...], w_ref[...], preferred_element_type=jnp.float32)
    logits = logits + b_ref[...]
    mx = jnp.max(logits, axis=-1, keepdims=True)
    ex = jnp.exp(logits - mx)
    o_ref[...] = ex / jnp.sum(ex, axis=-1, keepdims=True)


def _fc_softmax(pooled, Wfc, bfc):
    k = pooled.shape[0]
    m = Wfc.shape[1]
    out = pl.pallas_call(
        _fc_softmax_kernel,
        in_specs=[
            pl.BlockSpec((1, k), lambda: (0, 0)),
            pl.BlockSpec((k, m), lambda: (0, 0)),
            pl.BlockSpec((1, m), lambda: (0, 0)),
        ],
        out_specs=pl.BlockSpec((1, m), lambda: (0, 0)),
        out_shape=jax.ShapeDtypeStruct((1, m), jnp.float32),
    )(pooled.reshape(1, k), Wfc, bfc.reshape(1, m))
    return out[0]


def _gcn(x, src, dst, W, b, dinv_norm, relu_out):
    h = _matmul_bias(x, W, jnp.zeros((W.shape[1],), jnp.float32))
    msg = h[src] * dinv_norm[:, None]
    out = jax.ops.segment_sum(msg, dst, num_segments=_N)
    out = out + b
    if relu_out:
        out = jnp.maximum(out, 0.0)
    return out


def kernel(x, W1, b1, W2, b2, Wg, att_src, att_dst, bg, Wfc, bfc, edge_index):
    loop = jnp.arange(_N, dtype=edge_index.dtype)
    src = jnp.concatenate([edge_index[0], loop])
    dst = jnp.concatenate([edge_index[1], loop])

    ones = jnp.ones(src.shape[0], dtype=jnp.float32)
    deg = jax.ops.segment_sum(ones, dst, num_segments=_N)
    dinv = jnp.where(deg > 0, jax.lax.rsqrt(jnp.maximum(deg, 1e-12)), 0.0)
    norm = dinv[src] * dinv[dst]

    h = _gcn(x, src, dst, W1, b1, norm, relu_out=True)
    h = _gcn(h, src, dst, W2, b2, norm, relu_out=True)

    hg, a_src, a_dst = _gat_proj(h, Wg, att_src, att_dst)
    hg = hg.reshape(_N, _HEADS, _GOUT)
    e = jax.nn.leaky_relu(a_src[src] + a_dst[dst], negative_slope=0.2)
    e_max = jax.ops.segment_max(e, dst, num_segments=_N)
    e_max = jnp.where(jnp.isfinite(e_max), e_max, 0.0)
    p = jnp.exp(e - e_max[dst])
    denom = jax.ops.segment_sum(p, dst, num_segments=_N)
    alpha = p / (denom[dst] + 1e-16)
    msg = hg[src] * alpha[:, :, None]
    out = jax.ops.segment_sum(msg, dst, num_segments=_N)
    hga = jnp.maximum(out.reshape(_N, _HEADS * _GOUT) + bg, 0.0)

    pooled = jnp.mean(hga, axis=0)
    return _fc_softmax(pooled, Wfc, bfc)
